# baseline (speedup 1.0000x reference)
import functools

import jax
import jax.numpy as jnp
from jax import lax
from jax.experimental import pallas as pl
from jax.experimental.pallas import tpu as pltpu
from jax.experimental.pallas import tpu_sc as plsc

_NCHUNKS = 8


def kernel(words, w_table, w_bias):
    B = words.shape[0]
    V, D = w_table.shape
    info = plsc.get_sparse_core_info()
    NC, NS = info.num_cores, info.num_subcores
    NW = NC * NS
    b_per_w = B // NW
    chunk = b_per_w // _NCHUNKS
    mesh = plsc.VectorSubcoreMesh(core_axis_name="c", subcore_axis_name="s")

    @functools.partial(
        pl.kernel,
        mesh=mesh,
        out_type=(
            jax.ShapeDtypeStruct((B, D), jnp.float32),
            jax.ShapeDtypeStruct((B,), jnp.float32),
        ),
        scratch_types=(
            [
                pltpu.VMEM((b_per_w,), jnp.int32),
                pltpu.VMEM((b_per_w,), jnp.float32),
            ]
            + [pltpu.VMEM((chunk, D), jnp.float32) for _ in range(_NCHUNKS)]
            + [pltpu.SemaphoreType.DMA for _ in range(_NCHUNKS)]
            + [pltpu.SemaphoreType.DMA, pltpu.SemaphoreType.DMA]
        ),
    )
    def glove_gather(words_hbm, table_hbm, bias_hbm, emb_hbm, bout_hbm,
                     idx_v, bias_v, *bufs_and_sems):
        bufs = bufs_and_sems[:_NCHUNKS]
        gsems = bufs_and_sems[_NCHUNKS:2 * _NCHUNKS]
        sem_out, sem_bias = bufs_and_sems[2 * _NCHUNKS:]
        wid = lax.axis_index("s") * NC + lax.axis_index("c")
        base = wid * b_per_w

        pltpu.sync_copy(words_hbm.at[pl.ds(base, b_per_w)], idx_v)
        c_bias = pltpu.async_copy(bias_hbm.at[idx_v], bias_v, sem_bias)
        gathers = [
            pltpu.async_copy(
                table_hbm.at[idx_v.at[pl.ds(c * chunk, chunk)]],
                bufs[c], gsems[c])
            for c in range(_NCHUNKS)
        ]
        outs = []
        for c in range(_NCHUNKS):
            gathers[c].wait()
            outs.append(pltpu.async_copy(
                bufs[c], emb_hbm.at[pl.ds(base + c * chunk, chunk)], sem_out))
        c_bias.wait()
        outs.append(pltpu.async_copy(
            bias_v, bout_hbm.at[pl.ds(base, b_per_w)], sem_bias))
        for o in outs:
            o.wait()

    emb, bias = glove_gather(words, w_table, w_bias.reshape(V))
    return emb, bias.reshape(B, 1)


# speedup vs baseline: 1.0012x; 1.0012x over previous
import functools

import jax
import jax.numpy as jnp
from jax import lax
from jax.experimental import pallas as pl
from jax.experimental.pallas import tpu as pltpu
from jax.experimental.pallas import tpu_sc as plsc

_NCHUNKS = 8


def kernel(words, w_table, w_bias):
    B = words.shape[0]
    V, D = w_table.shape
    info = plsc.get_sparse_core_info()
    NC, NS = info.num_cores, info.num_subcores
    NW = NC * NS
    b_per_w = B // NW
    chunk = b_per_w // _NCHUNKS
    mesh = plsc.VectorSubcoreMesh(core_axis_name="c", subcore_axis_name="s")

    @functools.partial(
        pl.kernel,
        mesh=mesh,
        out_type=(
            jax.ShapeDtypeStruct((B, D), jnp.float32),
            jax.ShapeDtypeStruct((B,), jnp.float32),
        ),
        scratch_types=(
            [
                pltpu.VMEM((b_per_w,), jnp.int32),
                pltpu.VMEM((b_per_w,), jnp.float32),
            ]
            + [pltpu.VMEM((chunk, D), jnp.float32) for _ in range(_NCHUNKS)]
            + [pltpu.SemaphoreType.DMA for _ in range(_NCHUNKS)]
            + [pltpu.SemaphoreType.DMA, pltpu.SemaphoreType.DMA]
        ),
    )
    def glove_gather(words_hbm, table_hbm, bias_hbm, emb_hbm, bout_hbm,
                     idx_v, bias_v, *bufs_and_sems):
        bufs = bufs_and_sems[:_NCHUNKS]
        gsems = bufs_and_sems[_NCHUNKS:2 * _NCHUNKS]
        sem_out, sem_bias = bufs_and_sems[2 * _NCHUNKS:]
        wid = lax.axis_index("s") * NC + lax.axis_index("c")
        base = wid * b_per_w

        half = b_per_w // 2
        c_idx2 = None
        pltpu.sync_copy(words_hbm.at[pl.ds(base, half)],
                        idx_v.at[pl.ds(0, half)])
        c_idx2 = pltpu.async_copy(words_hbm.at[pl.ds(base + half, half)],
                                  idx_v.at[pl.ds(half, half)], sem_out)
        gathers = []
        for c in range(_NCHUNKS // 2):
            gathers.append(pltpu.async_copy(
                table_hbm.at[idx_v.at[pl.ds(c * chunk, chunk)]],
                bufs[c], gsems[c]))
        c_idx2.wait()
        c_bias = pltpu.async_copy(bias_hbm.at[idx_v], bias_v, sem_bias)
        for c in range(_NCHUNKS // 2, _NCHUNKS):
            gathers.append(pltpu.async_copy(
                table_hbm.at[idx_v.at[pl.ds(c * chunk, chunk)]],
                bufs[c], gsems[c]))
        outs = []
        for c in range(_NCHUNKS):
            gathers[c].wait()
            outs.append(pltpu.async_copy(
                bufs[c], emb_hbm.at[pl.ds(base + c * chunk, chunk)], sem_out))
        c_bias.wait()
        outs.append(pltpu.async_copy(
            bias_v, bout_hbm.at[pl.ds(base, b_per_w)], sem_bias))
        for o in outs:
            o.wait()

    emb, bias = glove_gather(words, w_table, w_bias.reshape(V))
    return emb, bias.reshape(B, 1)
